# baseline (speedup 1.0000x reference)
import functools

import jax
import jax.numpy as jnp
from jax import lax
from jax.experimental import pallas as pl
from jax.experimental.pallas import tpu as pltpu
from jax.experimental.pallas import tpu_sc as plsc

_N_USERS = 50000
_N = 100000
_D = 32
_E = 1600000
_B = 4096
_L = 3

_NC = 2
_NS = 16
_H = _N // _NC
_HP = 50176
_DUMMY = _HP

_E_PAD = 1638400
_BLOCKS_PER_SUB = 800
_GB = 5
_N_GROUPS = _BLOCKS_PER_SUB // _GB

_ROWS_PER_SUB = _HP // _NS
_ZROWS = 224

_G_TOT = 3 * _B
_G_BLKS = _G_TOT // 128
_G_PER_W = _G_BLKS // (_NC * _NS)


def _spmm_body(cur_h, src_h, dst_h, ev_h, out_h,
               src_v, dst_v, ev_v, rows_v, acc_sh, gsem, ssem0):
    c = lax.axis_index("c")
    s = lax.axis_index("s")
    zero = jnp.zeros((16,), jnp.float32)

    def zb(i, carry):
        rows_v[i, pl.ds(0, 16)] = zero
        rows_v[i, pl.ds(16, 16)] = zero
        return carry

    lax.fori_loop(0, _ZROWS, zb, 0)
    strip = s * _ROWS_PER_SUB
    zstage = rows_v.at[pl.ds(0, _ZROWS)]
    for k in range(_ROWS_PER_SUB // _ZROWS):
        pltpu.sync_copy(zstage, acc_sh.at[pl.ds(strip + k * _ZROWS, _ZROWS)])
    plsc.subcore_barrier()

    off = c * _H

    def group_body(g, carry):
        base = (s * _BLOCKS_PER_SUB + g * _GB) * 128
        pltpu.sync_copy(src_h.at[pl.ds(base, _GB * 128)], src_v)
        pltpu.sync_copy(dst_h.at[pl.ds(base, _GB * 128)], dst_v)
        pltpu.sync_copy(ev_h.at[pl.ds(base, _GB * 128)], ev_v)

        pltpu.async_copy(cur_h.at[src_v], rows_v, gsem).wait()

        def scale_body(v, carry2):
            col = v * 16
            ev16 = ev_v[pl.ds(col, 16)]
            loc = dst_v[pl.ds(col, 16)] - off
            ok = (loc >= 0) & (loc < _H)
            dst_v[pl.ds(col, 16)] = jnp.where(
                ok, loc, jnp.full((16,), _DUMMY, jnp.int32))
            base_row = v * 16
            for l in range(16):
                eb = jnp.full((16,), ev16[l], jnp.float32)
                r = base_row + l
                rows_v[r, pl.ds(0, 16)] = rows_v[r, pl.ds(0, 16)] * eb
                rows_v[r, pl.ds(16, 16)] = rows_v[r, pl.ds(16, 16)] * eb
            return carry2

        lax.fori_loop(0, _GB * 8, scale_body, 0)

        pltpu.async_copy(rows_v, acc_sh.at[dst_v], ssem0, add=True).wait()
        return carry

    lax.fori_loop(0, _N_GROUPS, group_body, 0)
    plsc.subcore_barrier()

    out_base = c * _HP + s * _ROWS_PER_SUB
    for k in range(_ROWS_PER_SUB // _ZROWS):
        pltpu.sync_copy(acc_sh.at[pl.ds(strip + k * _ZROWS, _ZROWS)], zstage)
        pltpu.sync_copy(zstage, out_h.at[pl.ds(out_base + k * _ZROWS, _ZROWS)])


@functools.partial(
    pl.kernel,
    out_type=jax.ShapeDtypeStruct((_NC * _HP, _D), jnp.float32),
    mesh=plsc.VectorSubcoreMesh(core_axis_name="c", subcore_axis_name="s"),
    scratch_types=[
        pltpu.VMEM((_GB * 128,), jnp.int32),
        pltpu.VMEM((_GB * 128,), jnp.int32),
        pltpu.VMEM((_GB * 128,), jnp.float32),
        pltpu.VMEM((_GB * 128, _D), jnp.float32),
        pltpu.VMEM_SHARED((_HP + 8, _D), jnp.float32),
        pltpu.SemaphoreType.DMA,
        pltpu.SemaphoreType.DMA,
    ],
    compiler_params=pltpu.CompilerParams(use_tc_tiling_on_sc=False),
)
def _spmm(cur_h, src_h, dst_h, ev_h, out_h,
          src_v, dst_v, ev_v, rows_v, acc_sh, gsem, ssem0):
    _spmm_body(cur_h, src_h, dst_h, ev_h, out_h,
               src_v, dst_v, ev_v, rows_v, acc_sh, gsem, ssem0)


@functools.partial(
    pl.kernel,
    out_type=jax.ShapeDtypeStruct((_G_TOT, _D), jnp.float32),
    mesh=plsc.VectorSubcoreMesh(core_axis_name="c", subcore_axis_name="s"),
    scratch_types=[
        pltpu.VMEM((_G_PER_W, 128), jnp.int32),
        pltpu.VMEM((_G_PER_W * 128, _D), jnp.float32),
        pltpu.SemaphoreType.DMA,
    ],
    compiler_params=pltpu.CompilerParams(use_tc_tiling_on_sc=False),
)
def _gather_rows(tab_h, idx_h, out_h, idx_v, rows_v, sem):
    c = lax.axis_index("c")
    s = lax.axis_index("s")
    w = s * _NC + c
    blk = w * _G_PER_W
    pltpu.sync_copy(idx_h.at[w], idx_v)
    for j in range(_G_PER_W):
        pltpu.async_copy(tab_h.at[idx_v.at[j]],
                         rows_v.at[pl.ds(j * 128, 128)], sem).wait()
    pltpu.sync_copy(rows_v, out_h.at[pl.ds(blk * 128, _G_PER_W * 128)])


_NORM_BLK = 2000


def _norm_body(scale, seg_ref, acc_ref, cur_ref, accout_ref):
    x = seg_ref[...]
    nrm = jnp.sqrt(jnp.sum(x * x, axis=1, keepdims=True))
    y = x / jnp.maximum(nrm, 1e-12)
    cur_ref[...] = y
    accout_ref[...] = (acc_ref[...] + y) * scale


def _norm_call(seg, acc, scale):
    bs = pl.BlockSpec((_NORM_BLK, _D), lambda i: (i, 0))
    return pl.pallas_call(
        functools.partial(_norm_body, scale),
        grid=(_N // _NORM_BLK,),
        in_specs=[bs, bs],
        out_specs=[bs, bs],
        out_shape=[jax.ShapeDtypeStruct((_N, _D), jnp.float32)] * 2,
    )(seg, acc)


def _loss_body(u_ref, p_ref, n_ref, o_ref):
    u = u_ref[...]
    d = jnp.sum(u * n_ref[...], axis=1, keepdims=True) \
        - jnp.sum(u * p_ref[...], axis=1, keepdims=True)
    sp = jnp.maximum(d, 0.0) + jnp.log(1.0 + jnp.exp(-jnp.abs(d)))
    o_ref[...] = (jnp.sum(sp) / _B).reshape(1, 1)


def _loss_call(u, p, n):
    return pl.pallas_call(
        _loss_body,
        out_shape=jax.ShapeDtypeStruct((1, 1), jnp.float32),
    )(u, p, n)


def kernel(user_id, pos_item, neg_item, edge_index, edge_values,
           user_weight, item_weight):
    cur = jnp.concatenate([user_weight, item_weight], axis=0)
    dst = edge_index[0]
    src = edge_index[1]

    pad = _E_PAD - _E
    src_p = jnp.concatenate([src, jnp.zeros((pad,), jnp.int32)])
    dst_p = jnp.concatenate([dst, jnp.full((pad,), _N, jnp.int32)])
    ev_p = jnp.concatenate([edge_values, jnp.zeros((pad,), jnp.float32)])
    src2 = src_p
    dst2 = dst_p
    ev2 = ev_p

    acc = cur
    for layer in range(_L):
        seg_p = _spmm(cur, src2, dst2, ev2)
        seg = jnp.concatenate([seg_p[:_H], seg_p[_HP:_HP + _H]], axis=0)
        scale = 0.25 if layer == _L - 1 else 1.0
        cur, acc = _norm_call(seg, acc, scale)

    all_embeddings = acc
    idx = jnp.concatenate([user_id, pos_item + _N_USERS, neg_item + _N_USERS])
    g = _gather_rows(all_embeddings,
                     idx.reshape(_NC * _NS, _G_PER_W, 128))
    u = g[:_B]
    p = g[_B:2 * _B]
    n = g[2 * _B:]
    rec_loss = _loss_call(u, p, n)[0, 0]
    return (rec_loss, all_embeddings)


# speedup vs baseline: 1.1899x; 1.1899x over previous
import functools

import jax
import jax.numpy as jnp
from jax import lax
from jax.experimental import pallas as pl
from jax.experimental.pallas import tpu as pltpu
from jax.experimental.pallas import tpu_sc as plsc

_N_USERS = 50000
_N = 100000
_D = 32
_E = 1600000
_B = 4096
_L = 3

_NC = 2
_NS = 16
_H = _N // _NC
_HP = 50176
_DUMMY = _HP

_E_PAD = 1638400
_G = 640

_NW = _NC * _NS
_PSLICE = _E_PAD // _NW
_SLOT = _PSLICE + 1280
_CIN = 2560
_STG = _G + 16

_ROWS_PER_SUB = _HP // _NS
_ZROWS = 224

_G_TOT = 3 * _B
_G_BLKS = _G_TOT // 128
_G_PER_W = _G_BLKS // (_NC * _NS)


def _part_side(stage_s, stage_d, stage_e, srcb_h, dstb_h, evb_h,
               s16, d16, e16, mask, obase, cur, out):
    csum = plsc.cumsum(jnp.where(mask, 1, 0).astype(jnp.int32))
    pos = cur + csum - 1
    plsc.store_scatter(stage_s, [pos], s16, mask=mask)
    plsc.store_scatter(stage_d, [pos], d16, mask=mask)
    plsc.store_scatter(stage_e, [pos], e16, mask=mask)
    n = csum[15]
    cur = cur + n
    flush = cur >= _G
    oat = obase + pl.multiple_of(out, _G)

    @pl.when(flush)
    def _():
        pltpu.sync_copy(stage_s.at[pl.ds(0, _G)],
                        srcb_h.at[pl.ds(oat, _G)])
        pltpu.sync_copy(stage_d.at[pl.ds(0, _G)],
                        dstb_h.at[pl.ds(oat, _G)])
        pltpu.sync_copy(stage_e.at[pl.ds(0, _G)],
                        evb_h.at[pl.ds(oat, _G)])
        stage_s[pl.ds(0, 16)] = stage_s[pl.ds(_G, 16)]
        stage_d[pl.ds(0, 16)] = stage_d[pl.ds(_G, 16)]
        stage_e[pl.ds(0, 16)] = stage_e[pl.ds(_G, 16)]

    cur = jnp.where(flush, cur - _G, cur)
    out = jnp.where(flush, out + _G, out)
    return cur, out, n


def _part_tail(stage_s, stage_d, stage_e, srcb_h, dstb_h, evb_h,
               obase, cur, out):
    iota = jnp.arange(16, dtype=jnp.int32)
    zi = jnp.zeros((16,), jnp.int32)
    zf = jnp.zeros((16,), jnp.float32)
    for k in range(_G // 16):
        keep = (iota + k * 16) < cur
        stage_s[pl.ds(k * 16, 16)] = jnp.where(
            keep, stage_s[pl.ds(k * 16, 16)], zi)
        stage_d[pl.ds(k * 16, 16)] = jnp.where(
            keep, stage_d[pl.ds(k * 16, 16)], zi)
        stage_e[pl.ds(k * 16, 16)] = jnp.where(
            keep, stage_e[pl.ds(k * 16, 16)], zf)

    oat = obase + pl.multiple_of(out, _G)

    @pl.when(cur > 0)
    def _():
        pltpu.sync_copy(stage_s.at[pl.ds(0, _G)],
                        srcb_h.at[pl.ds(oat, _G)])
        pltpu.sync_copy(stage_d.at[pl.ds(0, _G)],
                        dstb_h.at[pl.ds(oat, _G)])
        pltpu.sync_copy(stage_e.at[pl.ds(0, _G)],
                        evb_h.at[pl.ds(oat, _G)])

    return (out + jnp.where(cur > 0, _G, 0)) // _G


@functools.partial(
    pl.kernel,
    out_type=[
        jax.ShapeDtypeStruct((2 * _NW * _SLOT,), jnp.int32),
        jax.ShapeDtypeStruct((2 * _NW * _SLOT,), jnp.int32),
        jax.ShapeDtypeStruct((2 * _NW * _SLOT,), jnp.float32),
        jax.ShapeDtypeStruct((_NW, 16), jnp.int32),
    ],
    mesh=plsc.VectorSubcoreMesh(core_axis_name="c", subcore_axis_name="s"),
    scratch_types=[
        pltpu.VMEM((_CIN,), jnp.int32),
        pltpu.VMEM((_CIN,), jnp.int32),
        pltpu.VMEM((_CIN,), jnp.float32),
        pltpu.VMEM((_STG,), jnp.int32),
        pltpu.VMEM((_STG,), jnp.int32),
        pltpu.VMEM((_STG,), jnp.float32),
        pltpu.VMEM((_STG,), jnp.int32),
        pltpu.VMEM((_STG,), jnp.int32),
        pltpu.VMEM((_STG,), jnp.float32),
        pltpu.VMEM((16,), jnp.int32),
    ],
    compiler_params=pltpu.CompilerParams(use_tc_tiling_on_sc=False,
                                         needs_layout_passes=False),
)
def _partition(src_h, dst_h, ev_h, srcb_h, dstb_h, evb_h, meta_h,
               in_s, in_d, in_e, sa_s, sa_d, sa_e, sb_s, sb_d, sb_e, meta_v):
    c = lax.axis_index("c")
    s = lax.axis_index("s")
    w = s * _NC + c
    ebase = w * _PSLICE
    obase_a = (2 * w) * _SLOT
    obase_b = (2 * w + 1) * _SLOT

    def chunk(k, carry):
        cur_a, out_a, cur_b, out_b = carry
        cb = ebase + k * _CIN
        pltpu.sync_copy(src_h.at[pl.ds(cb, _CIN)], in_s)
        pltpu.sync_copy(dst_h.at[pl.ds(cb, _CIN)], in_d)
        pltpu.sync_copy(ev_h.at[pl.ds(cb, _CIN)], in_e)

        def vec(i, carry2):
            cur_a, out_a, cur_b, out_b = carry2
            col = i * 16
            s16 = in_s[pl.ds(col, 16)]
            d16 = in_d[pl.ds(col, 16)]
            e16 = in_e[pl.ds(col, 16)]
            m_a = d16 < _H
            cur_a, out_a, _ = _part_side(
                sa_s, sa_d, sa_e, srcb_h, dstb_h, evb_h,
                s16, d16, e16, m_a, obase_a, cur_a, out_a)
            cur_b, out_b, _ = _part_side(
                sb_s, sb_d, sb_e, srcb_h, dstb_h, evb_h,
                s16, d16 - _H, e16, jnp.logical_not(m_a),
                obase_b, cur_b, out_b)
            return (cur_a, out_a, cur_b, out_b)

        return lax.fori_loop(0, _CIN // 16, vec, carry)

    cur_a, out_a, cur_b, out_b = lax.fori_loop(
        0, _PSLICE // _CIN, chunk, (0, 0, 0, 0))
    ng_a = _part_tail(sa_s, sa_d, sa_e, srcb_h, dstb_h, evb_h,
                      obase_a, cur_a, out_a)
    ng_b = _part_tail(sb_s, sb_d, sb_e, srcb_h, dstb_h, evb_h,
                      obase_b, cur_b, out_b)
    iota = jnp.arange(16, dtype=jnp.int32)
    meta_v[pl.ds(0, 16)] = jnp.where(
        iota == 0, ng_a, jnp.where(iota == 1, ng_b, 0))
    pltpu.sync_copy(meta_v, meta_h.at[w])


def _spmm_body(cur_h, srcb_h, dstb_h, evb_h, meta_h, out_h,
               src_v, dst_v, ev_v, rows_v, meta_v, acc_sh, gsem, ssem0):
    c = lax.axis_index("c")
    s = lax.axis_index("s")
    zero = jnp.zeros((16,), jnp.float32)

    def zb(i, carry):
        rows_v[i, pl.ds(0, 16)] = zero
        rows_v[i, pl.ds(16, 16)] = zero
        return carry

    lax.fori_loop(0, _ZROWS, zb, 0)
    strip = s * _ROWS_PER_SUB
    zstage = rows_v.at[pl.ds(0, _ZROWS)]
    for k in range(_ROWS_PER_SUB // _ZROWS):
        pltpu.sync_copy(zstage, acc_sh.at[pl.ds(strip + k * _ZROWS, _ZROWS)])
    plsc.subcore_barrier()

    for t in range(2):
        wslot = 2 * s + t
        pltpu.sync_copy(meta_h.at[wslot], meta_v)
        mv = meta_v[pl.ds(0, 16)]
        ng = jnp.where(c == 0, mv[0], mv[1])
        base = (2 * wslot + c) * _SLOT

        def group_body(g, carry):
            ebase = base + g * _G
            pltpu.sync_copy(srcb_h.at[pl.ds(ebase, _G)], src_v)
            pltpu.sync_copy(dstb_h.at[pl.ds(ebase, _G)], dst_v)
            pltpu.sync_copy(evb_h.at[pl.ds(ebase, _G)], ev_v)

            pltpu.async_copy(cur_h.at[src_v], rows_v, gsem).wait()

            def scale_body(v, carry2):
                col = v * 16
                ev16 = ev_v[pl.ds(col, 16)]
                base_row = v * 16
                for l in range(16):
                    eb = jnp.full((16,), ev16[l], jnp.float32)
                    r = base_row + l
                    rows_v[r, pl.ds(0, 16)] = rows_v[r, pl.ds(0, 16)] * eb
                    rows_v[r, pl.ds(16, 16)] = rows_v[r, pl.ds(16, 16)] * eb
                return carry2

            lax.fori_loop(0, _G // 16, scale_body, 0)

            pltpu.async_copy(rows_v, acc_sh.at[dst_v], ssem0, add=True).wait()
            return carry

        lax.fori_loop(0, ng, group_body, 0)
    plsc.subcore_barrier()

    out_base = c * _HP + s * _ROWS_PER_SUB
    for k in range(_ROWS_PER_SUB // _ZROWS):
        pltpu.sync_copy(acc_sh.at[pl.ds(strip + k * _ZROWS, _ZROWS)], zstage)
        pltpu.sync_copy(zstage, out_h.at[pl.ds(out_base + k * _ZROWS, _ZROWS)])


@functools.partial(
    pl.kernel,
    out_type=jax.ShapeDtypeStruct((_NC * _HP, _D), jnp.float32),
    mesh=plsc.VectorSubcoreMesh(core_axis_name="c", subcore_axis_name="s"),
    scratch_types=[
        pltpu.VMEM((_G,), jnp.int32),
        pltpu.VMEM((_G,), jnp.int32),
        pltpu.VMEM((_G,), jnp.float32),
        pltpu.VMEM((_G, _D), jnp.float32),
        pltpu.VMEM((16,), jnp.int32),
        pltpu.VMEM_SHARED((_HP + 8, _D), jnp.float32),
        pltpu.SemaphoreType.DMA,
        pltpu.SemaphoreType.DMA,
    ],
    compiler_params=pltpu.CompilerParams(use_tc_tiling_on_sc=False),
)
def _spmm(cur_h, srcb_h, dstb_h, evb_h, meta_h, out_h,
          src_v, dst_v, ev_v, rows_v, meta_v, acc_sh, gsem, ssem0):
    _spmm_body(cur_h, srcb_h, dstb_h, evb_h, meta_h, out_h,
               src_v, dst_v, ev_v, rows_v, meta_v, acc_sh, gsem, ssem0)


@functools.partial(
    pl.kernel,
    out_type=jax.ShapeDtypeStruct((_G_TOT, _D), jnp.float32),
    mesh=plsc.VectorSubcoreMesh(core_axis_name="c", subcore_axis_name="s"),
    scratch_types=[
        pltpu.VMEM((_G_PER_W, 128), jnp.int32),
        pltpu.VMEM((_G_PER_W * 128, _D), jnp.float32),
        pltpu.SemaphoreType.DMA,
    ],
    compiler_params=pltpu.CompilerParams(use_tc_tiling_on_sc=False),
)
def _gather_rows(tab_h, idx_h, out_h, idx_v, rows_v, sem):
    c = lax.axis_index("c")
    s = lax.axis_index("s")
    w = s * _NC + c
    blk = w * _G_PER_W
    pltpu.sync_copy(idx_h.at[w], idx_v)
    for j in range(_G_PER_W):
        pltpu.async_copy(tab_h.at[idx_v.at[j]],
                         rows_v.at[pl.ds(j * 128, 128)], sem).wait()
    pltpu.sync_copy(rows_v, out_h.at[pl.ds(blk * 128, _G_PER_W * 128)])


_NORM_BLK = 2000


def _norm_body(scale, seg_ref, acc_ref, cur_ref, accout_ref):
    x = seg_ref[...]
    nrm = jnp.sqrt(jnp.sum(x * x, axis=1, keepdims=True))
    y = x / jnp.maximum(nrm, 1e-12)
    cur_ref[...] = y
    accout_ref[...] = (acc_ref[...] + y) * scale


def _norm_call(seg, acc, scale):
    bs = pl.BlockSpec((_NORM_BLK, _D), lambda i: (i, 0))
    return pl.pallas_call(
        functools.partial(_norm_body, scale),
        grid=(_N // _NORM_BLK,),
        in_specs=[bs, bs],
        out_specs=[bs, bs],
        out_shape=[jax.ShapeDtypeStruct((_N, _D), jnp.float32)] * 2,
    )(seg, acc)


def _loss_body(u_ref, p_ref, n_ref, o_ref):
    u = u_ref[...]
    d = jnp.sum(u * n_ref[...], axis=1, keepdims=True) \
        - jnp.sum(u * p_ref[...], axis=1, keepdims=True)
    sp = jnp.maximum(d, 0.0) + jnp.log(1.0 + jnp.exp(-jnp.abs(d)))
    o_ref[...] = (jnp.sum(sp) / _B).reshape(1, 1)


def _loss_call(u, p, n):
    return pl.pallas_call(
        _loss_body,
        out_shape=jax.ShapeDtypeStruct((1, 1), jnp.float32),
    )(u, p, n)


def kernel(user_id, pos_item, neg_item, edge_index, edge_values,
           user_weight, item_weight):
    cur = jnp.concatenate([user_weight, item_weight], axis=0)
    dst = edge_index[0]
    src = edge_index[1]

    pad = _E_PAD - _E
    src_p = jnp.concatenate([src, jnp.zeros((pad,), jnp.int32)])
    dst_p = jnp.concatenate([dst, jnp.zeros((pad,), jnp.int32)])
    ev_p = jnp.concatenate([edge_values, jnp.zeros((pad,), jnp.float32)])

    srcb, dstb, evb, meta = _partition(src_p, dst_p, ev_p)

    acc = cur
    for layer in range(_L):
        seg_p = _spmm(cur, srcb, dstb, evb, meta)
        seg = jnp.concatenate([seg_p[:_H], seg_p[_HP:_HP + _H]], axis=0)
        scale = 0.25 if layer == _L - 1 else 1.0
        cur, acc = _norm_call(seg, acc, scale)

    all_embeddings = acc
    idx = jnp.concatenate([user_id, pos_item + _N_USERS, neg_item + _N_USERS])
    g = _gather_rows(all_embeddings,
                     idx.reshape(_NC * _NS, _G_PER_W, 128))
    u = g[:_B]
    p = g[_B:2 * _B]
    n = g[2 * _B:]
    rec_loss = _loss_call(u, p, n)[0, 0]
    return (rec_loss, all_embeddings)


# speedup vs baseline: 1.1900x; 1.0001x over previous
import functools

import jax
import jax.numpy as jnp
from jax import lax
from jax.experimental import pallas as pl
from jax.experimental.pallas import tpu as pltpu
from jax.experimental.pallas import tpu_sc as plsc

_N_USERS = 50000
_N = 100000
_D = 32
_E = 1600000
_B = 4096
_L = 3

_NC = 2
_NS = 16
_H = _N // _NC
_HP = 50176

_E_PAD = 1638400
_G = 640

_NW = _NC * _NS
_PSLICE = _E_PAD // _NW
_SLOT = _PSLICE + 1280
_CIN = 2560
_STG = _G + 16

_ROWS_PER_SUB = _HP // _NS
_ZROWS = 224

_G_TOT = 3 * _B
_G_BLKS = _G_TOT // 128
_G_PER_W = _G_BLKS // (_NC * _NS)


def _part_side(stage_s, stage_d, stage_e, srcb_h, dstb_h, evb_h,
               s16, d16, e16, mask, obase, cur, out):
    csum = plsc.cumsum(jnp.where(mask, 1, 0).astype(jnp.int32))
    pos = cur + csum - 1
    plsc.store_scatter(stage_s, [pos], s16, mask=mask)
    plsc.store_scatter(stage_d, [pos], d16, mask=mask)
    plsc.store_scatter(stage_e, [pos], e16, mask=mask)
    n = csum[15]
    cur = cur + n
    flush = cur >= _G
    oat = obase + pl.multiple_of(out, _G)

    @pl.when(flush)
    def _():
        pltpu.sync_copy(stage_s.at[pl.ds(0, _G)],
                        srcb_h.at[pl.ds(oat, _G)])
        pltpu.sync_copy(stage_d.at[pl.ds(0, _G)],
                        dstb_h.at[pl.ds(oat, _G)])
        pltpu.sync_copy(stage_e.at[pl.ds(0, _G)],
                        evb_h.at[pl.ds(oat, _G)])
        stage_s[pl.ds(0, 16)] = stage_s[pl.ds(_G, 16)]
        stage_d[pl.ds(0, 16)] = stage_d[pl.ds(_G, 16)]
        stage_e[pl.ds(0, 16)] = stage_e[pl.ds(_G, 16)]

    cur = jnp.where(flush, cur - _G, cur)
    out = jnp.where(flush, out + _G, out)
    return cur, out, n


def _part_tail(stage_s, stage_d, stage_e, srcb_h, dstb_h, evb_h,
               obase, cur, out):
    iota = jnp.arange(16, dtype=jnp.int32)
    zi = jnp.zeros((16,), jnp.int32)
    zf = jnp.zeros((16,), jnp.float32)
    for k in range(_G // 16):
        keep = (iota + k * 16) < cur
        stage_s[pl.ds(k * 16, 16)] = jnp.where(
            keep, stage_s[pl.ds(k * 16, 16)], zi)
        stage_d[pl.ds(k * 16, 16)] = jnp.where(
            keep, stage_d[pl.ds(k * 16, 16)], zi)
        stage_e[pl.ds(k * 16, 16)] = jnp.where(
            keep, stage_e[pl.ds(k * 16, 16)], zf)

    oat = obase + pl.multiple_of(out, _G)

    @pl.when(cur > 0)
    def _():
        pltpu.sync_copy(stage_s.at[pl.ds(0, _G)],
                        srcb_h.at[pl.ds(oat, _G)])
        pltpu.sync_copy(stage_d.at[pl.ds(0, _G)],
                        dstb_h.at[pl.ds(oat, _G)])
        pltpu.sync_copy(stage_e.at[pl.ds(0, _G)],
                        evb_h.at[pl.ds(oat, _G)])

    return (out + jnp.where(cur > 0, _G, 0)) // _G


@functools.partial(
    pl.kernel,
    out_type=[
        jax.ShapeDtypeStruct((2 * _NW * _SLOT,), jnp.int32),
        jax.ShapeDtypeStruct((2 * _NW * _SLOT,), jnp.int32),
        jax.ShapeDtypeStruct((2 * _NW * _SLOT,), jnp.float32),
        jax.ShapeDtypeStruct((_NW, 16), jnp.int32),
    ],
    mesh=plsc.VectorSubcoreMesh(core_axis_name="c", subcore_axis_name="s"),
    scratch_types=[
        pltpu.VMEM((_CIN,), jnp.int32),
        pltpu.VMEM((_CIN,), jnp.int32),
        pltpu.VMEM((_CIN,), jnp.float32),
        pltpu.VMEM((_STG,), jnp.int32),
        pltpu.VMEM((_STG,), jnp.int32),
        pltpu.VMEM((_STG,), jnp.float32),
        pltpu.VMEM((_STG,), jnp.int32),
        pltpu.VMEM((_STG,), jnp.int32),
        pltpu.VMEM((_STG,), jnp.float32),
        pltpu.VMEM((16,), jnp.int32),
    ],
    compiler_params=pltpu.CompilerParams(use_tc_tiling_on_sc=False,
                                         needs_layout_passes=False),
)
def _partition(src_h, dst_h, ev_h, srcb_h, dstb_h, evb_h, meta_h,
               in_s, in_d, in_e, sa_s, sa_d, sa_e, sb_s, sb_d, sb_e, meta_v):
    c = lax.axis_index("c")
    s = lax.axis_index("s")
    w = s * _NC + c
    ebase = w * _PSLICE
    obase_a = (2 * w) * _SLOT
    obase_b = (2 * w + 1) * _SLOT

    def chunk(k, carry):
        cur_a, out_a, cur_b, out_b = carry
        cb = ebase + k * _CIN
        pltpu.sync_copy(src_h.at[pl.ds(cb, _CIN)], in_s)
        pltpu.sync_copy(dst_h.at[pl.ds(cb, _CIN)], in_d)
        pltpu.sync_copy(ev_h.at[pl.ds(cb, _CIN)], in_e)

        def vec(i, carry2):
            cur_a, out_a, cur_b, out_b = carry2
            col = i * 16
            s16 = in_s[pl.ds(col, 16)]
            d16 = in_d[pl.ds(col, 16)]
            e16 = in_e[pl.ds(col, 16)]
            m_a = d16 < _H
            cur_a, out_a, _ = _part_side(
                sa_s, sa_d, sa_e, srcb_h, dstb_h, evb_h,
                s16, d16, e16, m_a, obase_a, cur_a, out_a)
            cur_b, out_b, _ = _part_side(
                sb_s, sb_d, sb_e, srcb_h, dstb_h, evb_h,
                s16, d16 - _H, e16, jnp.logical_not(m_a),
                obase_b, cur_b, out_b)
            return (cur_a, out_a, cur_b, out_b)

        return lax.fori_loop(0, _CIN // 16, vec, carry)

    cur_a, out_a, cur_b, out_b = lax.fori_loop(
        0, _PSLICE // _CIN, chunk, (0, 0, 0, 0))
    ng_a = _part_tail(sa_s, sa_d, sa_e, srcb_h, dstb_h, evb_h,
                      obase_a, cur_a, out_a)
    ng_b = _part_tail(sb_s, sb_d, sb_e, srcb_h, dstb_h, evb_h,
                      obase_b, cur_b, out_b)
    iota = jnp.arange(16, dtype=jnp.int32)
    meta_v[pl.ds(0, 16)] = jnp.where(
        iota == 0, ng_a, jnp.where(iota == 1, ng_b, 0))
    pltpu.sync_copy(meta_v, meta_h.at[w])


def _spmm_body(cur_h, srcb_h, dstb_h, evb_h, meta_h, out_h,
               src_v, dst_v, ev_v, rows_v, meta_v, acc_sh, gsem, ssem0):
    c = lax.axis_index("c")
    s = lax.axis_index("s")
    zero = jnp.zeros((16,), jnp.float32)

    def zb(i, carry):
        rows_v[i, pl.ds(0, 16)] = zero
        rows_v[i, pl.ds(16, 16)] = zero
        return carry

    lax.fori_loop(0, _ZROWS, zb, 0)
    strip = s * _ROWS_PER_SUB
    zstage = rows_v.at[pl.ds(0, _ZROWS)]
    for k in range(_ROWS_PER_SUB // _ZROWS):
        pltpu.sync_copy(zstage, acc_sh.at[pl.ds(strip + k * _ZROWS, _ZROWS)])
    plsc.subcore_barrier()

    for t in range(2):
        wslot = 2 * s + t
        pltpu.sync_copy(meta_h.at[wslot], meta_v)
        mv = meta_v[pl.ds(0, 16)]
        ng = jnp.where(c == 0, mv[0], mv[1])
        base = (2 * wslot + c) * _SLOT

        def group_body(g, carry):
            ebase = base + g * _G
            pltpu.sync_copy(srcb_h.at[pl.ds(ebase, _G)], src_v)
            pltpu.sync_copy(dstb_h.at[pl.ds(ebase, _G)], dst_v)
            pltpu.sync_copy(evb_h.at[pl.ds(ebase, _G)], ev_v)

            pltpu.async_copy(cur_h.at[src_v], rows_v, gsem).wait()

            def scale_body(v, carry2):
                col = v * 16
                ev16 = ev_v[pl.ds(col, 16)]
                base_row = v * 16
                for l in range(16):
                    eb = jnp.full((16,), ev16[l], jnp.float32)
                    r = base_row + l
                    rows_v[r, pl.ds(0, 16)] = rows_v[r, pl.ds(0, 16)] * eb
                    rows_v[r, pl.ds(16, 16)] = rows_v[r, pl.ds(16, 16)] * eb
                return carry2

            lax.fori_loop(0, _G // 16, scale_body, 0)

            pltpu.async_copy(rows_v, acc_sh.at[dst_v], ssem0, add=True).wait()
            return carry

        lax.fori_loop(0, ng, group_body, 0)
    plsc.subcore_barrier()

    out_base = c * _HP + s * _ROWS_PER_SUB
    for k in range(_ROWS_PER_SUB // _ZROWS):
        pltpu.sync_copy(acc_sh.at[pl.ds(strip + k * _ZROWS, _ZROWS)], zstage)
        pltpu.sync_copy(zstage, out_h.at[pl.ds(out_base + k * _ZROWS, _ZROWS)])


@functools.partial(
    pl.kernel,
    out_type=jax.ShapeDtypeStruct((_NC * _HP, _D), jnp.float32),
    mesh=plsc.VectorSubcoreMesh(core_axis_name="c", subcore_axis_name="s"),
    scratch_types=[
        pltpu.VMEM((_G,), jnp.int32),
        pltpu.VMEM((_G,), jnp.int32),
        pltpu.VMEM((_G,), jnp.float32),
        pltpu.VMEM((_G, _D), jnp.float32),
        pltpu.VMEM((16,), jnp.int32),
        pltpu.VMEM_SHARED((_HP + 8, _D), jnp.float32),
        pltpu.SemaphoreType.DMA,
        pltpu.SemaphoreType.DMA,
    ],
    compiler_params=pltpu.CompilerParams(use_tc_tiling_on_sc=False),
)
def _spmm(cur_h, srcb_h, dstb_h, evb_h, meta_h, out_h,
          src_v, dst_v, ev_v, rows_v, meta_v, acc_sh, gsem, ssem0):
    _spmm_body(cur_h, srcb_h, dstb_h, evb_h, meta_h, out_h,
               src_v, dst_v, ev_v, rows_v, meta_v, acc_sh, gsem, ssem0)


@functools.partial(
    pl.kernel,
    out_type=jax.ShapeDtypeStruct((_G_TOT, _D), jnp.float32),
    mesh=plsc.VectorSubcoreMesh(core_axis_name="c", subcore_axis_name="s"),
    scratch_types=[
        pltpu.VMEM((_G_PER_W, 128), jnp.int32),
        pltpu.VMEM((_G_PER_W * 128, _D), jnp.float32),
        pltpu.SemaphoreType.DMA,
    ],
    compiler_params=pltpu.CompilerParams(use_tc_tiling_on_sc=False),
)
def _gather_rows(tab_h, idx_h, out_h, idx_v, rows_v, sem):
    c = lax.axis_index("c")
    s = lax.axis_index("s")
    w = s * _NC + c
    blk = w * _G_PER_W
    pltpu.sync_copy(idx_h.at[w], idx_v)
    for j in range(_G_PER_W):
        pltpu.async_copy(tab_h.at[idx_v.at[j]],
                         rows_v.at[pl.ds(j * 128, 128)], sem).wait()
    pltpu.sync_copy(rows_v, out_h.at[pl.ds(blk * 128, _G_PER_W * 128)])


_NORM_BLK = 2000


def _norm_body(scale, seg_ref, acc_ref, cur_ref, accout_ref):
    x = seg_ref[...]
    nrm = jnp.sqrt(jnp.sum(x * x, axis=1, keepdims=True))
    y = x / jnp.maximum(nrm, 1e-12)
    cur_ref[...] = y
    accout_ref[...] = (acc_ref[...] + y) * scale


def _norm_call(seg, acc, scale):
    bs = pl.BlockSpec((_NORM_BLK, _D), lambda i: (i, 0))
    return pl.pallas_call(
        functools.partial(_norm_body, scale),
        grid=(_N // _NORM_BLK,),
        in_specs=[bs, bs],
        out_specs=[bs, bs],
        out_shape=[jax.ShapeDtypeStruct((_N, _D), jnp.float32)] * 2,
    )(seg, acc)


def _loss_body(u_ref, p_ref, n_ref, o_ref):
    u = u_ref[...]
    d = jnp.sum(u * n_ref[...], axis=1, keepdims=True) \
        - jnp.sum(u * p_ref[...], axis=1, keepdims=True)
    sp = jnp.maximum(d, 0.0) + jnp.log(1.0 + jnp.exp(-jnp.abs(d)))
    o_ref[...] = (jnp.sum(sp) / _B).reshape(1, 1)


def _loss_call(u, p, n):
    return pl.pallas_call(
        _loss_body,
        out_shape=jax.ShapeDtypeStruct((1, 1), jnp.float32),
    )(u, p, n)


def kernel(user_id, pos_item, neg_item, edge_index, edge_values,
           user_weight, item_weight):
    cur = jnp.concatenate([user_weight, item_weight], axis=0)
    dst = edge_index[0]
    src = edge_index[1]

    pad = _E_PAD - _E
    src_p = jnp.concatenate([src, jnp.zeros((pad,), jnp.int32)])
    dst_p = jnp.concatenate([dst, jnp.zeros((pad,), jnp.int32)])
    ev_p = jnp.concatenate([edge_values, jnp.zeros((pad,), jnp.float32)])

    srcb, dstb, evb, meta = _partition(src_p, dst_p, ev_p)

    acc = cur
    for layer in range(_L):
        seg_p = _spmm(cur, srcb, dstb, evb, meta)
        seg = jnp.concatenate([seg_p[:_H], seg_p[_HP:_HP + _H]], axis=0)
        scale = 0.25 if layer == _L - 1 else 1.0
        cur, acc = _norm_call(seg, acc, scale)

    all_embeddings = acc
    idx = jnp.concatenate([user_id, pos_item + _N_USERS, neg_item + _N_USERS])
    g = _gather_rows(all_embeddings,
                     idx.reshape(_NC * _NS, _G_PER_W, 128))
    u = g[:_B]
    p = g[_B:2 * _B]
    n = g[2 * _B:]
    rec_loss = _loss_call(u, p, n)[0, 0]
    return (rec_loss, all_embeddings)
